# Initial kernel scaffold; baseline (speedup 1.0000x reference)
import jax
import jax.numpy as jnp
from jax.experimental import pallas as pl


def kernel(x, edge_index, W1, b1, W2, b2, head_w, head_b):
    raise NotImplementedError("write your pallas kernel here")


# speedup vs baseline: 19.9235x; 19.9235x over previous
import functools

import jax
import jax.numpy as jnp
from jax import lax
from jax.experimental import pallas as pl
from jax.experimental.pallas import tpu as pltpu
from jax.experimental.pallas import tpu_sc as plsc

N = 10000
D = 128
E = 320000

NC = 2
NS = 16
NW = NC * NS
C = 128
M = -(-E // (NW * C))
EPAD = NW * M * C
NACC = 10112
RT = NACC // NS
NDUM = NACC - N

_mesh = plsc.VectorSubcoreMesh(core_axis_name="c", subcore_axis_name="s")

DW = 128


def _make_deg(dw):
    @functools.partial(
        pl.kernel,
        out_type=jax.ShapeDtypeStruct((NC, NACC, dw), jnp.float32),
        mesh=_mesh,
        scratch_types=[
            pltpu.VMEM((M, C), jnp.int32),
            pltpu.VMEM((C, dw), jnp.float32),
            pltpu.VMEM_SHARED((NACC, dw), jnp.float32),
        ],
    )
    def _deg(dstb, zcol, ocol, degp, idx_d, ones_v, acc_sh):
        c = lax.axis_index("c")
        s = lax.axis_index("s")
        w = c * NS + s
        r0 = s * RT
        pltpu.sync_copy(zcol.at[pl.ds(r0, RT)], acc_sh.at[pl.ds(r0, RT)])
        pltpu.sync_copy(ocol, ones_v)
        pltpu.sync_copy(dstb.at[w], idx_d)
        plsc.subcore_barrier()

        def body(j, carry):
            pltpu.sync_copy(ones_v, acc_sh.at[idx_d.at[j]], add=True)
            return carry

        lax.fori_loop(0, M, body, 0)
        plsc.subcore_barrier()
        pltpu.sync_copy(acc_sh.at[pl.ds(r0, RT)], degp.at[c, pl.ds(r0, RT)])

    return _deg


_sc_deg = _make_deg(DW)


@functools.partial(
    pl.kernel,
    out_type=jax.ShapeDtypeStruct((NC, NACC, D), jnp.float32),
    mesh=_mesh,
    scratch_types=[
        pltpu.VMEM((M, C), jnp.int32),
        pltpu.VMEM((M, C), jnp.int32),
        pltpu.VMEM((C, D), jnp.float32),
        pltpu.VMEM_SHARED((NACC, D), jnp.float32),
        pltpu.SemaphoreType.DMA,
    ],
)
def _sc_agg(g, srcb, dstb, zmat, accp, idx_s, idx_d, rows, acc_sh, gsem):
    c = lax.axis_index("c")
    s = lax.axis_index("s")
    w = c * NS + s
    r0 = s * RT
    pltpu.sync_copy(zmat.at[pl.ds(r0, RT)], acc_sh.at[pl.ds(r0, RT)])
    pltpu.sync_copy(srcb.at[w], idx_s)
    pltpu.sync_copy(dstb.at[w], idx_d)
    plsc.subcore_barrier()

    def body(j, carry):
        pltpu.async_copy(g.at[idx_s.at[j]], rows, gsem).wait()
        pltpu.sync_copy(rows, acc_sh.at[idx_d.at[j]], add=True)
        return carry

    lax.fori_loop(0, M, body, 0)
    plsc.subcore_barrier()
    pltpu.sync_copy(acc_sh.at[pl.ds(r0, RT)], accp.at[c, pl.ds(r0, RT)])


_R = 1024
_G = -(-NACC // _R)


def _prep_body(x_ref, w_ref, degp_ref, g_ref, dis_ref):
    deg = degp_ref[0, :, 0:1] + degp_ref[1, :, 0:1] + 1.0
    dis = lax.rsqrt(deg)
    h = jnp.dot(x_ref[...], w_ref[...], preferred_element_type=jnp.float32)
    g_ref[...] = h * dis
    dis_ref[...] = dis


@jax.jit
def _tc_prep(x, W1, degp):
    return pl.pallas_call(
        _prep_body,
        grid=(_G,),
        in_specs=[
            pl.BlockSpec((_R, D), lambda i: (i, 0)),
            pl.BlockSpec((D, D), lambda i: (0, 0)),
            pl.BlockSpec((NC, _R, DW), lambda i: (0, i, 0)),
        ],
        out_specs=[
            pl.BlockSpec((_R, D), lambda i: (i, 0)),
            pl.BlockSpec((_R, 1), lambda i: (i, 0)),
        ],
        out_shape=[
            jax.ShapeDtypeStruct((N, D), jnp.float32),
            jax.ShapeDtypeStruct((NACC, 1), jnp.float32),
        ],
    )(x, W1, degp)


def _mid_body(accp_ref, g1_ref, dis_ref, b_ref, w_ref, g2_ref):
    a = accp_ref[0] + accp_ref[1] + g1_ref[...]
    z = jnp.maximum(dis_ref[...] * a + b_ref[...], 0.0)
    g2_ref[...] = jnp.dot(z, w_ref[...],
                          preferred_element_type=jnp.float32) * dis_ref[...]


@jax.jit
def _tc_mid(accp, g1, dis, b1, W2):
    return pl.pallas_call(
        _mid_body,
        grid=(_G,),
        in_specs=[
            pl.BlockSpec((NC, _R, D), lambda i: (0, i, 0)),
            pl.BlockSpec((_R, D), lambda i: (i, 0)),
            pl.BlockSpec((_R, 1), lambda i: (i, 0)),
            pl.BlockSpec((1, D), lambda i: (0, 0)),
            pl.BlockSpec((D, D), lambda i: (0, 0)),
        ],
        out_specs=pl.BlockSpec((_R, D), lambda i: (i, 0)),
        out_shape=jax.ShapeDtypeStruct((N, D), jnp.float32),
    )(accp, g1, dis, b1, W2)


def _head_body(accp_ref, g2_ref, dis_ref, b_ref, hw_ref, hb_ref, o_ref):
    a = accp_ref[0] + accp_ref[1] + g2_ref[...]
    z = jnp.maximum(dis_ref[...] * a + b_ref[...], 0.0)
    o_ref[...] = jnp.dot(z, hw_ref[...],
                         preferred_element_type=jnp.float32) + hb_ref[...]


@jax.jit
def _tc_head(accp, g2, dis, b2, head_w, head_b):
    return pl.pallas_call(
        _head_body,
        grid=(_G,),
        in_specs=[
            pl.BlockSpec((NC, _R, D), lambda i: (0, i, 0)),
            pl.BlockSpec((_R, D), lambda i: (i, 0)),
            pl.BlockSpec((_R, 1), lambda i: (i, 0)),
            pl.BlockSpec((1, D), lambda i: (0, 0)),
            pl.BlockSpec((D, 1), lambda i: (0, 0)),
            pl.BlockSpec((1, 1), lambda i: (0, 0)),
        ],
        out_specs=pl.BlockSpec((_R, 1), lambda i: (i, 0)),
        out_shape=jax.ShapeDtypeStruct((N, 1), jnp.float32),
    )(accp, g2, dis, b2, head_w, head_b)


def kernel(x, edge_index, W1, b1, W2, b2, head_w, head_b):
    src, dst = edge_index[0], edge_index[1]
    npad = EPAD - E
    pad_i = jnp.arange(npad, dtype=jnp.int32)
    srcb = jnp.concatenate([src, pad_i % N]).reshape(NW, M, C)
    dstb = jnp.concatenate([dst, N + (pad_i % NDUM)]).reshape(NW, M, C)

    zcol = jnp.zeros((NACC, DW), jnp.float32)
    ocol = jnp.ones((C, DW), jnp.float32)
    zmat = jnp.zeros((NACC, D), jnp.float32)

    degp = _sc_deg(dstb, zcol, ocol)
    g1, dis = _tc_prep(x, W1, degp)
    accp1 = _sc_agg(g1, srcb, dstb, zmat)
    g2 = _tc_mid(accp1, g1, dis, b1.reshape(1, D), W2)
    accp2 = _sc_agg(g2, srcb, dstb, zmat)
    out = _tc_head(accp2, g2, dis, b2.reshape(1, D), head_w,
                   head_b.reshape(1, 1))
    return out[:, 0]


# speedup vs baseline: 26.8147x; 1.3459x over previous
import functools

import jax
import jax.numpy as jnp
from jax import lax
from jax.experimental import pallas as pl
from jax.experimental.pallas import tpu as pltpu
from jax.experimental.pallas import tpu_sc as plsc

N = 10000
D = 128
E = 320000

NC = 2
NS = 16
NW = NC * NS
C = 128
M = 80
MH = M // 2
EPAD = NW * M * C
NACC = 10112
RT = NACC // NS
NDUM = NACC - N

_mesh = plsc.VectorSubcoreMesh(core_axis_name="c", subcore_axis_name="s")

DW = 128


def _make_deg(dw):
    @functools.partial(
        pl.kernel,
        out_type=jax.ShapeDtypeStruct((NC, NACC, dw), jnp.float32),
        mesh=_mesh,
        scratch_types=[
            pltpu.VMEM((M, C), jnp.int32),
            pltpu.VMEM((C, dw), jnp.float32),
            pltpu.VMEM_SHARED((NACC, dw), jnp.float32),
        ],
    )
    def _deg(dstb, zcol, ocol, degp, idx_d, ones_v, acc_sh):
        c = lax.axis_index("c")
        s = lax.axis_index("s")
        w = c * NS + s
        r0 = s * RT
        pltpu.sync_copy(zcol.at[pl.ds(r0, RT)], acc_sh.at[pl.ds(r0, RT)])
        pltpu.sync_copy(ocol, ones_v)
        pltpu.sync_copy(dstb.at[w], idx_d)
        plsc.subcore_barrier()

        def body(j, carry):
            pltpu.sync_copy(ones_v, acc_sh.at[idx_d.at[j]], add=True)
            return carry

        lax.fori_loop(0, M, body, 0)
        plsc.subcore_barrier()
        pltpu.sync_copy(acc_sh.at[pl.ds(r0, RT)], degp.at[c, pl.ds(r0, RT)])

    return _deg


_sc_deg = _make_deg(DW)


@functools.partial(
    pl.kernel,
    out_type=jax.ShapeDtypeStruct((NC, NACC, D), jnp.float32),
    mesh=_mesh,
    scratch_types=[
        pltpu.VMEM((MH, C), jnp.int32),
        pltpu.VMEM((MH, C), jnp.int32),
        pltpu.VMEM((C, D), jnp.float32),
        pltpu.VMEM((C, D), jnp.float32),
        pltpu.VMEM_SHARED((NACC, D), jnp.float32),
        pltpu.SemaphoreType.DMA,
        pltpu.SemaphoreType.DMA,
    ],
)
def _sc_agg(g, srcb, dstb, zmat, accp, idx_s, idx_d, rows0, rows1, acc_sh,
            sem0, sem1):
    c = lax.axis_index("c")
    s = lax.axis_index("s")
    w = c * NS + s
    r0 = s * RT
    pltpu.sync_copy(zmat.at[pl.ds(r0, RT)], acc_sh.at[pl.ds(r0, RT)])
    plsc.subcore_barrier()

    for h in range(2):
        pltpu.sync_copy(srcb.at[w, pl.ds(h * MH, MH)], idx_s)
        pltpu.sync_copy(dstb.at[w, pl.ds(h * MH, MH)], idx_d)
        pltpu.async_copy(g.at[idx_s.at[0]], rows0, sem0)
        pltpu.async_copy(g.at[idx_s.at[1]], rows1, sem1)

        def body(i, carry):
            for (rows, sem, off) in ((rows0, sem0, 0), (rows1, sem1, 1)):
                j = 2 * i + off
                pltpu.make_async_copy(g.at[idx_s.at[j]], rows, sem).wait()
                pltpu.sync_copy(rows, acc_sh.at[idx_d.at[j]], add=True)

                @pl.when(j + 2 < MH)
                def _():
                    pltpu.async_copy(g.at[idx_s.at[j + 2]], rows, sem)
            return carry

        lax.fori_loop(0, MH // 2, body, 0)
    plsc.subcore_barrier()
    pltpu.sync_copy(acc_sh.at[pl.ds(r0, RT)], accp.at[c, pl.ds(r0, RT)])


_R = 1024
_G = -(-NACC // _R)


def _prep_body(x_ref, w_ref, degp_ref, g_ref, dis_ref):
    deg = degp_ref[0, :, 0:1] + degp_ref[1, :, 0:1] + 1.0
    dis = lax.rsqrt(deg)
    h = jnp.dot(x_ref[...], w_ref[...], preferred_element_type=jnp.float32)
    g_ref[...] = h * dis
    dis_ref[...] = dis


@jax.jit
def _tc_prep(x, W1, degp):
    return pl.pallas_call(
        _prep_body,
        grid=(_G,),
        in_specs=[
            pl.BlockSpec((_R, D), lambda i: (i, 0)),
            pl.BlockSpec((D, D), lambda i: (0, 0)),
            pl.BlockSpec((NC, _R, DW), lambda i: (0, i, 0)),
        ],
        out_specs=[
            pl.BlockSpec((_R, D), lambda i: (i, 0)),
            pl.BlockSpec((_R, 1), lambda i: (i, 0)),
        ],
        out_shape=[
            jax.ShapeDtypeStruct((N, D), jnp.float32),
            jax.ShapeDtypeStruct((NACC, 1), jnp.float32),
        ],
    )(x, W1, degp)


def _mid_body(accp_ref, g1_ref, dis_ref, b_ref, w_ref, g2_ref):
    a = accp_ref[0] + accp_ref[1] + g1_ref[...]
    z = jnp.maximum(dis_ref[...] * a + b_ref[...], 0.0)
    g2_ref[...] = jnp.dot(z, w_ref[...],
                          preferred_element_type=jnp.float32) * dis_ref[...]


@jax.jit
def _tc_mid(accp, g1, dis, b1, W2):
    return pl.pallas_call(
        _mid_body,
        grid=(_G,),
        in_specs=[
            pl.BlockSpec((NC, _R, D), lambda i: (0, i, 0)),
            pl.BlockSpec((_R, D), lambda i: (i, 0)),
            pl.BlockSpec((_R, 1), lambda i: (i, 0)),
            pl.BlockSpec((1, D), lambda i: (0, 0)),
            pl.BlockSpec((D, D), lambda i: (0, 0)),
        ],
        out_specs=pl.BlockSpec((_R, D), lambda i: (i, 0)),
        out_shape=jax.ShapeDtypeStruct((N, D), jnp.float32),
    )(accp, g1, dis, b1, W2)


def _head_body(accp_ref, g2_ref, dis_ref, b_ref, hw_ref, hb_ref, o_ref):
    a = accp_ref[0] + accp_ref[1] + g2_ref[...]
    z = jnp.maximum(dis_ref[...] * a + b_ref[...], 0.0)
    o_ref[...] = jnp.dot(z, hw_ref[...],
                         preferred_element_type=jnp.float32) + hb_ref[...]


@jax.jit
def _tc_head(accp, g2, dis, b2, head_w, head_b):
    return pl.pallas_call(
        _head_body,
        grid=(_G,),
        in_specs=[
            pl.BlockSpec((NC, _R, D), lambda i: (0, i, 0)),
            pl.BlockSpec((_R, D), lambda i: (i, 0)),
            pl.BlockSpec((_R, 1), lambda i: (i, 0)),
            pl.BlockSpec((1, D), lambda i: (0, 0)),
            pl.BlockSpec((D, 1), lambda i: (0, 0)),
            pl.BlockSpec((1, 1), lambda i: (0, 0)),
        ],
        out_specs=pl.BlockSpec((_R, 1), lambda i: (i, 0)),
        out_shape=jax.ShapeDtypeStruct((N, 1), jnp.float32),
    )(accp, g2, dis, b2, head_w, head_b)


def kernel(x, edge_index, W1, b1, W2, b2, head_w, head_b):
    src, dst = edge_index[0], edge_index[1]
    npad = EPAD - E
    pad_i = jnp.arange(npad, dtype=jnp.int32)
    srcb = jnp.concatenate([src, pad_i % N]).reshape(NW, M, C)
    dstb = jnp.concatenate([dst, N + (pad_i % NDUM)]).reshape(NW, M, C)

    zcol = jnp.zeros((NACC, DW), jnp.float32)
    ocol = jnp.ones((C, DW), jnp.float32)
    zmat = jnp.zeros((NACC, D), jnp.float32)

    degp = _sc_deg(dstb, zcol, ocol)
    g1, dis = _tc_prep(x, W1, degp)
    accp1 = _sc_agg(g1, srcb, dstb, zmat)
    g2 = _tc_mid(accp1, g1, dis, b1.reshape(1, D), W2)
    accp2 = _sc_agg(g2, srcb, dstb, zmat)
    out = _tc_head(accp2, g2, dis, b2.reshape(1, D), head_w,
                   head_b.reshape(1, 1))
    return out[:, 0]
